# baseline (speedup 1.0000x reference)
import jax
import jax.numpy as jnp
from jax.experimental import pallas as pl
from jax.experimental.pallas import tpu as pltpu
from jax.experimental.pallas import tpu_sc as plsc

_LANES = 16


def kernel(x, emb_table):
    B, S, D = x.shape
    pos = emb_table[:S]
    S_BLK = 4
    grid = (S // S_BLK,)

    vector_mesh = plsc.VectorSubcoreMesh(
        core_axis_name="core", subcore_axis_name="subcore"
    )

    @pl.kernel(out_type=jax.ShapeDtypeStruct((B, S, D), x.dtype),
               mesh=vector_mesh)
    def sc_add(x_hbm, emb_hbm, o_hbm):
        def body(x_vmem, emb_vmem, o_vmem):
            for r in range(S_BLK):
                @plsc.parallel_loop(0, D, step=_LANES, unroll=16)
                def _(c, r=r):
                    e = emb_vmem.at[r, pl.ds(c, _LANES)][...]
                    for b in range(B):
                        o_vmem.at[b, r, pl.ds(c, _LANES)][...] = (
                            x_vmem.at[b, r, pl.ds(c, _LANES)][...] + e
                        )

        pltpu.emit_pipeline(
            body,
            grid=grid,
            in_specs=[
                pl.BlockSpec((B, S_BLK, D), lambda i: (0, i, 0)),
                pl.BlockSpec((S_BLK, D), lambda i: (i, 0)),
            ],
            out_specs=[pl.BlockSpec((B, S_BLK, D), lambda i: (0, i, 0))],
            core_axis_name=("core", "subcore"),
            dimension_semantics=(pltpu.PARALLEL,),
        )(x_hbm, emb_hbm, o_hbm)

    return sc_add(x, pos)


# speedup vs baseline: 1.5452x; 1.5452x over previous
import jax
import jax.numpy as jnp
from jax.experimental import pallas as pl


def _add_kernel(x_ref, emb_ref, out_ref):
    out_ref[...] = x_ref[...] + emb_ref[...][None, :, :]


def kernel(x, emb_table):
    B, S, D = x.shape
    pos = emb_table[:S]
    S_BLK = 512
    grid = (S // S_BLK,)
    return pl.pallas_call(
        _add_kernel,
        grid=grid,
        in_specs=[
            pl.BlockSpec((B, S_BLK, D), lambda i: (0, i, 0)),
            pl.BlockSpec((S_BLK, D), lambda i: (i, 0)),
        ],
        out_specs=pl.BlockSpec((B, S_BLK, D), lambda i: (0, i, 0)),
        out_shape=jax.ShapeDtypeStruct((B, S, D), x.dtype),
    )(x, pos)
